# Initial kernel scaffold; baseline (speedup 1.0000x reference)
import jax
import jax.numpy as jnp
from jax.experimental import pallas as pl


def kernel(nodes, edges, senders, receivers, We1, be1, We2, be2, Wn1, bn1, Wn2, bn2):
    raise NotImplementedError("write your pallas kernel here")


# speedup vs baseline: 2.2214x; 2.2214x over previous
import functools

import jax
import jax.numpy as jnp
from jax import lax
from jax.experimental import pallas as pl
from jax.experimental.pallas import tpu as pltpu
from jax.experimental.pallas import tpu_sc as plsc

N = 10000
E = 160000
D = 256
DE = 16
H = 512

NC = 2
NS = 16
NW = NC * NS

_MESH = lambda: plsc.VectorSubcoreMesh(
    core_axis_name="c", subcore_axis_name="s", num_cores=NC, num_subcores=NS)

EPW = E // NW
GB = 200


def _sc_gather(nodes, senders, receivers):
  @functools.partial(
      pl.kernel,
      out_type=(jax.ShapeDtypeStruct((E, D), jnp.float32),
                jax.ShapeDtypeStruct((E, D), jnp.float32)),
      mesh=_MESH(),
      scratch_types=[
          pltpu.VMEM((GB,), jnp.int32),
          pltpu.VMEM((GB, D), jnp.float32),
          pltpu.SemaphoreType.DMA,
      ],
  )
  def k(nodes_hbm, snd_hbm, rcv_hbm, gs_hbm, gr_hbm, idx_v, rows_v, sem):
    wid = lax.axis_index("s") * NC + lax.axis_index("c")
    base = wid * EPW

    def body(i, carry):
      off = base + i * GB
      pltpu.sync_copy(snd_hbm.at[pl.ds(off, GB)], idx_v)
      pltpu.async_copy(nodes_hbm.at[idx_v], rows_v, sem).wait()
      pltpu.sync_copy(rows_v, gs_hbm.at[pl.ds(off, GB)])
      pltpu.sync_copy(rcv_hbm.at[pl.ds(off, GB)], idx_v)
      pltpu.async_copy(nodes_hbm.at[idx_v], rows_v, sem).wait()
      pltpu.sync_copy(rows_v, gr_hbm.at[pl.ds(off, GB)])
      return carry

    lax.fori_loop(0, EPW // GB, body, 0)

  return k(nodes, senders, receivers)


EPT = E // NS
SB = 200
NPAD = 10240
NPT = NPAD // NS
DH = D // NC
NLAST = N - (NS - 1) * NPT


def _sc_scatter(ue, receivers, zeros_half):
  @functools.partial(
      pl.kernel,
      out_type=jax.ShapeDtypeStruct((N, D), jnp.float32),
      mesh=_MESH(),
      scratch_types=[
          pltpu.VMEM((SB,), jnp.int32),
          pltpu.VMEM((SB, DH), jnp.float32),
          pltpu.VMEM_SHARED((NPAD, DH), jnp.float32),
      ],
  )
  def k(ue_hbm, rcv_hbm, zero_hbm, agg_hbm, idx_v, rows_v, acc_sh):
    c = lax.axis_index("c")
    s = lax.axis_index("s")
    col = pl.multiple_of(c * DH, DH)
    pltpu.sync_copy(zero_hbm, acc_sh.at[pl.ds(s * NPT, NPT)])
    plsc.subcore_barrier()

    ebase = s * EPT

    def body(i, carry):
      off = ebase + i * SB
      pltpu.sync_copy(rcv_hbm.at[pl.ds(off, SB)], idx_v)
      pltpu.sync_copy(ue_hbm.at[pl.ds(off, SB), pl.ds(col, DH)], rows_v)
      pltpu.sync_copy(rows_v, acc_sh.at[idx_v], add=True)
      return carry

    lax.fori_loop(0, EPT // SB, body, 0)
    plsc.subcore_barrier()
    @pl.when(s < NS - 1)
    def _():
      pltpu.sync_copy(acc_sh.at[pl.ds(s * NPT, NPT)],
                      agg_hbm.at[pl.ds(s * NPT, NPT), pl.ds(col, DH)])

    @pl.when(s == NS - 1)
    def _():
      pltpu.sync_copy(acc_sh.at[pl.ds((NS - 1) * NPT, NLAST)],
                      agg_hbm.at[pl.ds((NS - 1) * NPT, NLAST),
                                 pl.ds(col, DH)])

  return k(ue, receivers, zeros_half)


BE = 800


def _tc_edge_mlp(gs, gr, ed, ws, wr, we, b1, w2, b2):
  def body(gs_r, gr_r, ed_r, ws_r, wr_r, we_r, b1_r, w2_r, b2_r, out_r):
    h = jnp.dot(gs_r[...], ws_r[...], preferred_element_type=jnp.float32)
    h = h + jnp.dot(gr_r[...], wr_r[...], preferred_element_type=jnp.float32)
    h = h + jnp.dot(ed_r[...], we_r[...], preferred_element_type=jnp.float32)
    h = jnp.maximum(h + b1_r[...], 0.0)
    out_r[...] = (jnp.dot(h, w2_r[...], preferred_element_type=jnp.float32)
                  + b2_r[...])

  full = lambda shape: pl.BlockSpec(shape, lambda i: (0, 0))
  return pl.pallas_call(
      body,
      grid=(E // BE,),
      in_specs=[
          pl.BlockSpec((BE, D), lambda i: (i, 0)),
          pl.BlockSpec((BE, D), lambda i: (i, 0)),
          pl.BlockSpec((BE, DE), lambda i: (i, 0)),
          full((D, H)),
          full((D, H)),
          full((DE, H)),
          full((1, H)),
          full((H, D)),
          full((1, D)),
      ],
      out_specs=pl.BlockSpec((BE, D), lambda i: (i, 0)),
      out_shape=jax.ShapeDtypeStruct((E, D), jnp.float32),
  )(gs, gr, ed, ws, wr, we, b1, w2, b2)


BN = 1000


def _tc_node_mlp(nodes, agg, w1a, w1b, b1, w2, b2):
  def body(n_r, a_r, w1a_r, w1b_r, b1_r, w2_r, b2_r, out_r):
    h = jnp.dot(n_r[...], w1a_r[...], preferred_element_type=jnp.float32)
    h = h + jnp.dot(a_r[...], w1b_r[...], preferred_element_type=jnp.float32)
    h = jnp.maximum(h + b1_r[...], 0.0)
    out_r[...] = (jnp.dot(h, w2_r[...], preferred_element_type=jnp.float32)
                  + b2_r[...])

  full = lambda shape: pl.BlockSpec(shape, lambda i: (0, 0))
  return pl.pallas_call(
      body,
      grid=(N // BN,),
      in_specs=[
          pl.BlockSpec((BN, D), lambda i: (i, 0)),
          pl.BlockSpec((BN, D), lambda i: (i, 0)),
          full((D, H)),
          full((D, H)),
          full((1, H)),
          full((H, D)),
          full((1, D)),
      ],
      out_specs=pl.BlockSpec((BN, D), lambda i: (i, 0)),
      out_shape=jax.ShapeDtypeStruct((N, D), jnp.float32),
  )(nodes, agg, w1a, w1b, b1, w2, b2)


def kernel(nodes, edges, senders, receivers,
           We1, be1, We2, be2, Wn1, bn1, Wn2, bn2):
  ws, wr, we = We1[:D], We1[D:2 * D], We1[2 * D:]
  gs, gr = _sc_gather(nodes, senders, receivers)
  ue = _tc_edge_mlp(gs, gr, edges, ws, wr, we,
                    be1.reshape(1, H), We2, be2.reshape(1, D))
  zeros_half = jnp.zeros((NPT, DH), jnp.float32)
  agg = _sc_scatter(ue, receivers, zeros_half)
  un = _tc_node_mlp(nodes, agg, Wn1[:D], Wn1[D:],
                    bn1.reshape(1, H), Wn2, bn2.reshape(1, D))
  return (un, ue)


# speedup vs baseline: 2.2454x; 1.0108x over previous
import functools

import jax
import jax.numpy as jnp
from jax import lax
from jax.experimental import pallas as pl
from jax.experimental.pallas import tpu as pltpu
from jax.experimental.pallas import tpu_sc as plsc

N = 10000
E = 160000
D = 256
DE = 16
H = 512

NC = 2
NS = 16
NW = NC * NS

_MESH = lambda: plsc.VectorSubcoreMesh(
    core_axis_name="c", subcore_axis_name="s", num_cores=NC, num_subcores=NS)

EPW = E // NW
GB = 200


def _sc_gather(nodes, senders, receivers):
  dt = nodes.dtype

  @functools.partial(
      pl.kernel,
      out_type=(jax.ShapeDtypeStruct((E, D), dt),
                jax.ShapeDtypeStruct((E, D), dt)),
      mesh=_MESH(),
      scratch_types=[
          pltpu.VMEM((GB,), jnp.int32),
          pltpu.VMEM((GB, D), dt),
          pltpu.SemaphoreType.DMA,
      ],
  )
  def k(nodes_hbm, snd_hbm, rcv_hbm, gs_hbm, gr_hbm, idx_v, rows_v, sem):
    wid = lax.axis_index("s") * NC + lax.axis_index("c")
    base = wid * EPW

    def body(i, carry):
      off = base + i * GB
      pltpu.sync_copy(snd_hbm.at[pl.ds(off, GB)], idx_v)
      pltpu.async_copy(nodes_hbm.at[idx_v], rows_v, sem).wait()
      pltpu.sync_copy(rows_v, gs_hbm.at[pl.ds(off, GB)])
      pltpu.sync_copy(rcv_hbm.at[pl.ds(off, GB)], idx_v)
      pltpu.async_copy(nodes_hbm.at[idx_v], rows_v, sem).wait()
      pltpu.sync_copy(rows_v, gr_hbm.at[pl.ds(off, GB)])
      return carry

    lax.fori_loop(0, EPW // GB, body, 0)

  return k(nodes, senders, receivers)


EPT = E // NS
SB = 200
NPAD = 10240
NPT = NPAD // NS
DH = D // NC
NLAST = N - (NS - 1) * NPT


def _sc_scatter(ue, receivers, zeros_half):
  @functools.partial(
      pl.kernel,
      out_type=jax.ShapeDtypeStruct((N, D), jnp.float32),
      mesh=_MESH(),
      scratch_types=[
          pltpu.VMEM((SB,), jnp.int32),
          pltpu.VMEM((SB, DH), jnp.float32),
          pltpu.VMEM_SHARED((NPAD, DH), jnp.float32),
      ],
  )
  def k(ue_hbm, rcv_hbm, zero_hbm, agg_hbm, idx_v, rows_v, acc_sh):
    c = lax.axis_index("c")
    s = lax.axis_index("s")
    col = pl.multiple_of(c * DH, DH)
    pltpu.sync_copy(zero_hbm, acc_sh.at[pl.ds(s * NPT, NPT)])
    plsc.subcore_barrier()

    ebase = s * EPT

    def body(i, carry):
      off = ebase + i * SB
      pltpu.sync_copy(rcv_hbm.at[pl.ds(off, SB)], idx_v)
      pltpu.sync_copy(ue_hbm.at[pl.ds(off, SB), pl.ds(col, DH)], rows_v)
      pltpu.sync_copy(rows_v, acc_sh.at[idx_v], add=True)
      return carry

    lax.fori_loop(0, EPT // SB, body, 0)
    plsc.subcore_barrier()
    @pl.when(s < NS - 1)
    def _():
      pltpu.sync_copy(acc_sh.at[pl.ds(s * NPT, NPT)],
                      agg_hbm.at[pl.ds(s * NPT, NPT), pl.ds(col, DH)])

    @pl.when(s == NS - 1)
    def _():
      pltpu.sync_copy(acc_sh.at[pl.ds((NS - 1) * NPT, NLAST)],
                      agg_hbm.at[pl.ds((NS - 1) * NPT, NLAST),
                                 pl.ds(col, DH)])

  return k(ue, receivers, zeros_half)


BE = 800


def _tc_edge_mlp(gs, gr, ed, ws, wr, we, b1, w2, b2):
  def body(gs_r, gr_r, ed_r, ws_r, wr_r, we_r, b1_r, w2_r, b2_r, out_r):
    h = jnp.dot(gs_r[...].astype(jnp.bfloat16), ws_r[...],
                preferred_element_type=jnp.float32)
    h = h + jnp.dot(gr_r[...].astype(jnp.bfloat16), wr_r[...],
                    preferred_element_type=jnp.float32)
    h = h + jnp.dot(ed_r[...], we_r[...], preferred_element_type=jnp.float32)
    h = jnp.maximum(h + b1_r[...], 0.0).astype(jnp.bfloat16)
    out_r[...] = (jnp.dot(h, w2_r[...], preferred_element_type=jnp.float32)
                  + b2_r[...])

  full = lambda shape: pl.BlockSpec(shape, lambda i: (0, 0))
  return pl.pallas_call(
      body,
      grid=(E // BE,),
      in_specs=[
          pl.BlockSpec((BE, D), lambda i: (i, 0)),
          pl.BlockSpec((BE, D), lambda i: (i, 0)),
          pl.BlockSpec((BE, DE), lambda i: (i, 0)),

          full((D, H)),
          full((D, H)),
          full((DE, H)),
          full((1, H)),
          full((H, D)),
          full((1, D)),
      ],
      out_specs=pl.BlockSpec((BE, D), lambda i: (i, 0)),
      out_shape=jax.ShapeDtypeStruct((E, D), jnp.float32),
  )(gs, gr, ed, ws, wr, we, b1, w2, b2)


BN = 1000


def _tc_node_mlp(nodes, agg, w1a, w1b, b1, w2, b2):
  def body(n_r, a_r, w1a_r, w1b_r, b1_r, w2_r, b2_r, out_r):
    h = jnp.dot(n_r[...], w1a_r[...], preferred_element_type=jnp.float32)
    h = h + jnp.dot(a_r[...].astype(jnp.bfloat16), w1b_r[...],
                    preferred_element_type=jnp.float32)
    h = jnp.maximum(h + b1_r[...], 0.0).astype(jnp.bfloat16)
    out_r[...] = (jnp.dot(h, w2_r[...], preferred_element_type=jnp.float32)
                  + b2_r[...])

  full = lambda shape: pl.BlockSpec(shape, lambda i: (0, 0))
  return pl.pallas_call(
      body,
      grid=(N // BN,),
      in_specs=[
          pl.BlockSpec((BN, D), lambda i: (i, 0)),
          pl.BlockSpec((BN, D), lambda i: (i, 0)),
          full((D, H)),
          full((D, H)),
          full((1, H)),
          full((H, D)),
          full((1, D)),
      ],
      out_specs=pl.BlockSpec((BN, D), lambda i: (i, 0)),
      out_shape=jax.ShapeDtypeStruct((N, D), jnp.float32),
  )(nodes, agg, w1a, w1b, b1, w2, b2)


def kernel(nodes, edges, senders, receivers,
           We1, be1, We2, be2, Wn1, bn1, Wn2, bn2):
  bf = jnp.bfloat16
  We1b, We2b = We1.astype(bf), We2.astype(bf)
  ws, wr, we = We1b[:D], We1b[D:2 * D], We1b[2 * D:]
  gs, gr = _sc_gather(nodes, senders, receivers)
  ue = _tc_edge_mlp(gs, gr, edges.astype(bf), ws, wr, we,
                    be1.reshape(1, H), We2b, be2.reshape(1, D))
  zeros_half = jnp.zeros((NPT, DH), jnp.float32)
  agg = _sc_scatter(ue, receivers, zeros_half)
  Wn1b, Wn2b = Wn1.astype(bf), Wn2.astype(bf)
  un = _tc_node_mlp(nodes.astype(bf), agg, Wn1b[:D], Wn1b[D:],
                    bn1.reshape(1, H), Wn2b, bn2.reshape(1, D))
  return (un, ue)


# speedup vs baseline: 2.2610x; 1.0070x over previous
import functools

import jax
import jax.numpy as jnp
from jax import lax
from jax.experimental import pallas as pl
from jax.experimental.pallas import tpu as pltpu
from jax.experimental.pallas import tpu_sc as plsc

N = 10000
E = 160000
D = 256
DE = 16
H = 512

NC = 2
NS = 16
NW = NC * NS

_MESH = lambda: plsc.VectorSubcoreMesh(
    core_axis_name="c", subcore_axis_name="s", num_cores=NC, num_subcores=NS)

EPW = E // NW
GB = 200


DP = D // 2


def _sc_gather(nodes_u, senders, receivers):
  @functools.partial(
      pl.kernel,
      out_type=(jax.ShapeDtypeStruct((E, DP), jnp.uint32),
                jax.ShapeDtypeStruct((E, DP), jnp.uint32)),
      mesh=_MESH(),
      scratch_types=[
          pltpu.VMEM((GB,), jnp.int32),
          pltpu.VMEM((GB, DP), jnp.uint32),
          pltpu.SemaphoreType.DMA,
      ],
  )
  def k(nodes_hbm, snd_hbm, rcv_hbm, gs_hbm, gr_hbm, idx_v, rows_v, sem):
    wid = lax.axis_index("s") * NC + lax.axis_index("c")
    base = wid * EPW

    def body(i, carry):
      off = base + i * GB
      pltpu.sync_copy(snd_hbm.at[pl.ds(off, GB)], idx_v)
      pltpu.async_copy(nodes_hbm.at[idx_v], rows_v, sem).wait()
      pltpu.sync_copy(rows_v, gs_hbm.at[pl.ds(off, GB)])
      pltpu.sync_copy(rcv_hbm.at[pl.ds(off, GB)], idx_v)
      pltpu.async_copy(nodes_hbm.at[idx_v], rows_v, sem).wait()
      pltpu.sync_copy(rows_v, gr_hbm.at[pl.ds(off, GB)])
      return carry

    lax.fori_loop(0, EPW // GB, body, 0)

  return k(nodes_u, senders, receivers)


EPT = E // NS
SB = 200
NPAD = 10240
NPT = NPAD // NS
DH = D // NC
NLAST = N - (NS - 1) * NPT


def _sc_scatter(ue, receivers, zeros_half):
  @functools.partial(
      pl.kernel,
      out_type=jax.ShapeDtypeStruct((N, D), jnp.float32),
      mesh=_MESH(),
      scratch_types=[
          pltpu.VMEM((SB,), jnp.int32),
          pltpu.VMEM((SB, DH), jnp.float32),
          pltpu.VMEM_SHARED((NPAD, DH), jnp.float32),
      ],
  )
  def k(ue_hbm, rcv_hbm, zero_hbm, agg_hbm, idx_v, rows_v, acc_sh):
    c = lax.axis_index("c")
    s = lax.axis_index("s")
    col = pl.multiple_of(c * DH, DH)
    pltpu.sync_copy(zero_hbm, acc_sh.at[pl.ds(s * NPT, NPT)])
    plsc.subcore_barrier()

    ebase = s * EPT

    def body(i, carry):
      off = ebase + i * SB
      pltpu.sync_copy(rcv_hbm.at[pl.ds(off, SB)], idx_v)
      pltpu.sync_copy(ue_hbm.at[pl.ds(off, SB), pl.ds(col, DH)], rows_v)
      pltpu.sync_copy(rows_v, acc_sh.at[idx_v], add=True)
      return carry

    lax.fori_loop(0, EPT // SB, body, 0)
    plsc.subcore_barrier()
    @pl.when(s < NS - 1)
    def _():
      pltpu.sync_copy(acc_sh.at[pl.ds(s * NPT, NPT)],
                      agg_hbm.at[pl.ds(s * NPT, NPT), pl.ds(col, DH)])

    @pl.when(s == NS - 1)
    def _():
      pltpu.sync_copy(acc_sh.at[pl.ds((NS - 1) * NPT, NLAST)],
                      agg_hbm.at[pl.ds((NS - 1) * NPT, NLAST),
                                 pl.ds(col, DH)])

  return k(ue, receivers, zeros_half)


BE = 800


def _tc_edge_mlp(gs, gr, ed, ws, wr, we, b1, w2, b2):
  def body(gs_r, gr_r, ed_r, ws_r, wr_r, we_r, b1_r, w2_r, b2_r, out_r):
    bf = jnp.bfloat16
    hi = jnp.uint32(0xFFFF0000)

    def unpack(u):
      ev = jax.lax.bitcast_convert_type(u << 16, jnp.float32).astype(bf)
      od = jax.lax.bitcast_convert_type(u & hi, jnp.float32).astype(bf)
      return ev, od

    gse, gso = unpack(gs_r[...])
    gre, gro = unpack(gr_r[...])
    h = jnp.dot(gse, ws_r[...][:DP], preferred_element_type=jnp.float32)
    h = h + jnp.dot(gso, ws_r[...][DP:], preferred_element_type=jnp.float32)
    h = h + jnp.dot(gre, wr_r[...][:DP], preferred_element_type=jnp.float32)
    h = h + jnp.dot(gro, wr_r[...][DP:], preferred_element_type=jnp.float32)
    h = h + jnp.dot(ed_r[...], we_r[...], preferred_element_type=jnp.float32)
    h = jnp.maximum(h + b1_r[...], 0.0).astype(bf)
    out_r[...] = (jnp.dot(h, w2_r[...], preferred_element_type=jnp.float32)
                  + b2_r[...])

  full = lambda shape: pl.BlockSpec(shape, lambda i: (0, 0))
  return pl.pallas_call(
      body,
      grid=(E // BE,),
      in_specs=[
          pl.BlockSpec((BE, DP), lambda i: (i, 0)),
          pl.BlockSpec((BE, DP), lambda i: (i, 0)),
          pl.BlockSpec((BE, DE), lambda i: (i, 0)),
          full((D, H)),
          full((D, H)),
          full((DE, H)),
          full((1, H)),
          full((H, D)),
          full((1, D)),
      ],
      out_specs=pl.BlockSpec((BE, D), lambda i: (i, 0)),
      out_shape=jax.ShapeDtypeStruct((E, D), jnp.float32),
  )(gs, gr, ed, ws, wr, we, b1, w2, b2)


BN = 1000


def _tc_node_mlp(nodes, agg, w1a, w1b, b1, w2, b2):
  def body(n_r, a_r, w1a_r, w1b_r, b1_r, w2_r, b2_r, out_r):
    h = jnp.dot(n_r[...], w1a_r[...], preferred_element_type=jnp.float32)
    h = h + jnp.dot(a_r[...].astype(jnp.bfloat16), w1b_r[...],
                    preferred_element_type=jnp.float32)
    h = jnp.maximum(h + b1_r[...], 0.0).astype(jnp.bfloat16)
    out_r[...] = (jnp.dot(h, w2_r[...], preferred_element_type=jnp.float32)
                  + b2_r[...])

  full = lambda shape: pl.BlockSpec(shape, lambda i: (0, 0))
  return pl.pallas_call(
      body,
      grid=(N // BN,),
      in_specs=[
          pl.BlockSpec((BN, D), lambda i: (i, 0)),
          pl.BlockSpec((BN, D), lambda i: (i, 0)),
          full((D, H)),
          full((D, H)),
          full((1, H)),
          full((H, D)),
          full((1, D)),
      ],
      out_specs=pl.BlockSpec((BN, D), lambda i: (i, 0)),
      out_shape=jax.ShapeDtypeStruct((N, D), jnp.float32),
  )(nodes, agg, w1a, w1b, b1, w2, b2)


def kernel(nodes, edges, senders, receivers,
           We1, be1, We2, be2, Wn1, bn1, Wn2, bn2):
  bf = jnp.bfloat16
  We1b, We2b = We1.astype(bf), We2.astype(bf)
  ws, wr, we = We1b[:D], We1b[D:2 * D], We1b[2 * D:]
  ws = jnp.concatenate([ws[0::2], ws[1::2]], axis=0)
  wr = jnp.concatenate([wr[0::2], wr[1::2]], axis=0)
  nodes_u = jax.lax.bitcast_convert_type(
      nodes.astype(bf).reshape(N, DP, 2), jnp.uint32)
  gs, gr = _sc_gather(nodes_u, senders, receivers)
  ue = _tc_edge_mlp(gs, gr, edges.astype(bf), ws, wr, we,
                    be1.reshape(1, H), We2b, be2.reshape(1, D))
  zeros_half = jnp.zeros((NPT, DH), jnp.float32)
  agg = _sc_scatter(ue, receivers, zeros_half)
  Wn1b, Wn2b = Wn1.astype(bf), Wn2.astype(bf)
  un = _tc_node_mlp(nodes.astype(bf), agg, Wn1b[:D], Wn1b[D:],
                    bn1.reshape(1, H), Wn2b, bn2.reshape(1, D))
  return (un, ue)


# speedup vs baseline: 2.6527x; 1.1733x over previous
import functools

import jax
import jax.numpy as jnp
from jax import lax
from jax.experimental import pallas as pl
from jax.experimental.pallas import tpu as pltpu
from jax.experimental.pallas import tpu_sc as plsc

N = 10000
E = 160000
D = 256
DE = 16
H = 512

NC = 2
NS = 16
NW = NC * NS

_MESH = lambda: plsc.VectorSubcoreMesh(
    core_axis_name="c", subcore_axis_name="s", num_cores=NC, num_subcores=NS)

EPW = E // NW
GB = 200


DP = D // 2


RING = 4
NCH = EPW // GB


def _sc_gather(nodes_u, senders, receivers):
  @functools.partial(
      pl.kernel,
      out_type=(jax.ShapeDtypeStruct((E, DP), jnp.uint32),
                jax.ShapeDtypeStruct((E, DP), jnp.uint32)),
      mesh=_MESH(),
      scratch_types=[
          pltpu.VMEM((EPW,), jnp.int32),
          pltpu.VMEM((EPW,), jnp.int32),
          pltpu.VMEM((RING, GB, DP), jnp.uint32),
          pltpu.SemaphoreType.DMA((RING,)),
          pltpu.SemaphoreType.DMA((RING,)),
      ],
  )
  def k(nodes_hbm, snd_hbm, rcv_hbm, gs_hbm, gr_hbm, idx_s, idx_r, buf,
        gsem, wsem):
    wid = lax.axis_index("s") * NC + lax.axis_index("c")
    base = wid * EPW
    pltpu.sync_copy(snd_hbm.at[pl.ds(base, EPW)], idx_s)
    pltpu.sync_copy(rcv_hbm.at[pl.ds(base, EPW)], idx_r)

    def run(idx_v, out_hbm):
      def g_desc(i):
        par = lax.rem(i, RING)
        return pltpu.make_async_copy(
            nodes_hbm.at[idx_v.at[pl.ds(i * GB, GB)]], buf.at[par],
            gsem.at[par])

      def w_desc(i):
        par = lax.rem(i, RING)
        return pltpu.make_async_copy(
            buf.at[par], out_hbm.at[pl.ds(base + i * GB, GB)], wsem.at[par])

      for j in range(RING):
        g_desc(j).start()

      def body(i, carry):
        g_desc(i).wait()
        w_desc(i).start()

        @pl.when(jnp.logical_and(i >= 1, i + RING - 1 < NCH))
        def _():
          w_desc(i - 1).wait()
          g_desc(i + RING - 1).start()

        return carry

      lax.fori_loop(0, NCH, body, 0)
      for j in range(RING):
        w_desc(NCH - RING + j).wait()

    run(idx_s, gs_hbm)
    run(idx_r, gr_hbm)

  return k(nodes_u, senders, receivers)


EPT = E // NS
SB = 80
NCHS = EPT // SB
NPAD = 10240
NPT = NPAD // NS
DH = D // NC
NLAST = N - (NS - 1) * NPT


def _sc_scatter(ue, receivers, zeros_half):
  @functools.partial(
      pl.kernel,
      out_type=jax.ShapeDtypeStruct((N, D), jnp.float32),
      mesh=_MESH(),
      scratch_types=[
          pltpu.VMEM((RING, SB), jnp.int32),
          pltpu.VMEM((RING, SB, DH), jnp.float32),
          pltpu.VMEM_SHARED((NPAD, DH), jnp.float32),
          pltpu.SemaphoreType.DMA((RING,)),
          pltpu.SemaphoreType.DMA((RING,)),
      ],
  )
  def k(ue_hbm, rcv_hbm, zero_hbm, agg_hbm, idx_v, rows_v, acc_sh, lsem,
        asem):
    c = lax.axis_index("c")
    s = lax.axis_index("s")
    col = pl.multiple_of(c * DH, DH)
    pltpu.sync_copy(zero_hbm, acc_sh.at[pl.ds(s * NPT, NPT)])
    plsc.subcore_barrier()
    ebase = s * EPT

    def idx_desc(i):
      par = lax.rem(i, RING)
      off = ebase + i * SB
      return pltpu.make_async_copy(rcv_hbm.at[pl.ds(off, SB)],
                                   idx_v.at[par], lsem.at[par])

    def rows_desc(i):
      par = lax.rem(i, RING)
      off = ebase + i * SB
      return pltpu.make_async_copy(
          ue_hbm.at[pl.ds(off, SB), pl.ds(col, DH)], rows_v.at[par],
          lsem.at[par])

    def add_start(i):
      par = lax.rem(i, RING)
      pltpu.async_copy(rows_v.at[par], acc_sh.at[idx_v.at[par]],
                       asem.at[par], add=True)

    def add_wait(i):
      par = lax.rem(i, RING)
      pltpu.make_async_copy(rows_v.at[par], acc_sh.at[idx_v.at[par]],
                            asem.at[par]).wait()

    def fire_load(i):
      idx_desc(i).start()
      rows_desc(i).start()

    for j in range(RING):
      fire_load(j)

    def body(i, carry):
      idx_desc(i).wait()
      rows_desc(i).wait()
      add_start(i)

      @pl.when(jnp.logical_and(i >= 1, i + RING - 1 < NCHS))
      def _():
        add_wait(i - 1)
        fire_load(i + RING - 1)

      return carry

    lax.fori_loop(0, NCHS, body, 0)
    for j in range(RING):
      add_wait(NCHS - RING + j)
    plsc.subcore_barrier()
    @pl.when(s < NS - 1)
    def _():
      pltpu.sync_copy(acc_sh.at[pl.ds(s * NPT, NPT)],
                      agg_hbm.at[pl.ds(s * NPT, NPT), pl.ds(col, DH)])

    @pl.when(s == NS - 1)
    def _():
      pltpu.sync_copy(acc_sh.at[pl.ds((NS - 1) * NPT, NLAST)],
                      agg_hbm.at[pl.ds((NS - 1) * NPT, NLAST),
                                 pl.ds(col, DH)])

  return k(ue, receivers, zeros_half)


BE = 800


def _tc_edge_mlp(gs, gr, ed, ws, wr, we, b1, w2, b2):
  def body(gs_r, gr_r, ed_r, ws_r, wr_r, we_r, b1_r, w2_r, b2_r, out_r):
    bf = jnp.bfloat16
    hi = jnp.uint32(0xFFFF0000)

    def unpack(u):
      ev = jax.lax.bitcast_convert_type(u << 16, jnp.float32).astype(bf)
      od = jax.lax.bitcast_convert_type(u & hi, jnp.float32).astype(bf)
      return ev, od

    gse, gso = unpack(gs_r[...])
    gre, gro = unpack(gr_r[...])
    h = jnp.dot(gse, ws_r[...][:DP], preferred_element_type=jnp.float32)
    h = h + jnp.dot(gso, ws_r[...][DP:], preferred_element_type=jnp.float32)
    h = h + jnp.dot(gre, wr_r[...][:DP], preferred_element_type=jnp.float32)
    h = h + jnp.dot(gro, wr_r[...][DP:], preferred_element_type=jnp.float32)
    h = h + jnp.dot(ed_r[...], we_r[...], preferred_element_type=jnp.float32)
    h = jnp.maximum(h + b1_r[...], 0.0).astype(bf)
    out_r[...] = (jnp.dot(h, w2_r[...], preferred_element_type=jnp.float32)
                  + b2_r[...])

  full = lambda shape: pl.BlockSpec(shape, lambda i: (0, 0))
  return pl.pallas_call(
      body,
      grid=(E // BE,),
      in_specs=[
          pl.BlockSpec((BE, DP), lambda i: (i, 0)),
          pl.BlockSpec((BE, DP), lambda i: (i, 0)),
          pl.BlockSpec((BE, DE), lambda i: (i, 0)),
          full((D, H)),
          full((D, H)),
          full((DE, H)),
          full((1, H)),
          full((H, D)),
          full((1, D)),
      ],
      out_specs=pl.BlockSpec((BE, D), lambda i: (i, 0)),
      out_shape=jax.ShapeDtypeStruct((E, D), jnp.float32),
  )(gs, gr, ed, ws, wr, we, b1, w2, b2)


BN = 1000


def _tc_node_mlp(nodes, agg, w1a, w1b, b1, w2, b2):
  def body(n_r, a_r, w1a_r, w1b_r, b1_r, w2_r, b2_r, out_r):
    h = jnp.dot(n_r[...], w1a_r[...], preferred_element_type=jnp.float32)
    h = h + jnp.dot(a_r[...].astype(jnp.bfloat16), w1b_r[...],
                    preferred_element_type=jnp.float32)
    h = jnp.maximum(h + b1_r[...], 0.0).astype(jnp.bfloat16)
    out_r[...] = (jnp.dot(h, w2_r[...], preferred_element_type=jnp.float32)
                  + b2_r[...])

  full = lambda shape: pl.BlockSpec(shape, lambda i: (0, 0))
  return pl.pallas_call(
      body,
      grid=(N // BN,),
      in_specs=[
          pl.BlockSpec((BN, D), lambda i: (i, 0)),
          pl.BlockSpec((BN, D), lambda i: (i, 0)),
          full((D, H)),
          full((D, H)),
          full((1, H)),
          full((H, D)),
          full((1, D)),
      ],
      out_specs=pl.BlockSpec((BN, D), lambda i: (i, 0)),
      out_shape=jax.ShapeDtypeStruct((N, D), jnp.float32),
  )(nodes, agg, w1a, w1b, b1, w2, b2)


def kernel(nodes, edges, senders, receivers,
           We1, be1, We2, be2, Wn1, bn1, Wn2, bn2):
  bf = jnp.bfloat16
  We1b, We2b = We1.astype(bf), We2.astype(bf)
  ws, wr, we = We1b[:D], We1b[D:2 * D], We1b[2 * D:]
  ws = jnp.concatenate([ws[0::2], ws[1::2]], axis=0)
  wr = jnp.concatenate([wr[0::2], wr[1::2]], axis=0)
  nodes_u = jax.lax.bitcast_convert_type(
      nodes.astype(bf).reshape(N, DP, 2), jnp.uint32)
  gs, gr = _sc_gather(nodes_u, senders, receivers)
  ue = _tc_edge_mlp(gs, gr, edges.astype(bf), ws, wr, we,
                    be1.reshape(1, H), We2b, be2.reshape(1, D))
  zeros_half = jnp.zeros((NPT, DH), jnp.float32)
  agg = _sc_scatter(ue, receivers, zeros_half)
  Wn1b, Wn2b = Wn1.astype(bf), Wn2.astype(bf)
  un = _tc_node_mlp(nodes.astype(bf), agg, Wn1b[:D], Wn1b[D:],
                    bn1.reshape(1, H), Wn2b, bn2.reshape(1, D))
  return (un, ue)


# speedup vs baseline: 3.0324x; 1.1431x over previous
import functools

import jax
import jax.numpy as jnp
from jax import lax
from jax.experimental import pallas as pl
from jax.experimental.pallas import tpu as pltpu
from jax.experimental.pallas import tpu_sc as plsc

N = 10000
E = 160000
D = 256
DE = 16
H = 512

NC = 2
NS = 16
NW = NC * NS

_MESH = lambda: plsc.VectorSubcoreMesh(
    core_axis_name="c", subcore_axis_name="s", num_cores=NC, num_subcores=NS)

EPW = E // NW
GB = 200


DP = D // 2


RING = 4
NCH = EPW // GB


def _sc_gather(nodes_u, senders, receivers):
  @functools.partial(
      pl.kernel,
      out_type=(jax.ShapeDtypeStruct((E, DP), jnp.uint32),
                jax.ShapeDtypeStruct((E, DP), jnp.uint32)),
      mesh=_MESH(),
      scratch_types=[
          pltpu.VMEM((EPW,), jnp.int32),
          pltpu.VMEM((EPW,), jnp.int32),
          pltpu.VMEM((RING, GB, DP), jnp.uint32),
          pltpu.SemaphoreType.DMA((RING,)),
          pltpu.SemaphoreType.DMA((RING,)),
      ],
  )
  def k(nodes_hbm, snd_hbm, rcv_hbm, gs_hbm, gr_hbm, idx_s, idx_r, buf,
        gsem, wsem):
    wid = lax.axis_index("s") * NC + lax.axis_index("c")
    base = wid * EPW
    pltpu.sync_copy(snd_hbm.at[pl.ds(base, EPW)], idx_s)
    pltpu.sync_copy(rcv_hbm.at[pl.ds(base, EPW)], idx_r)

    def run(idx_v, out_hbm):
      def g_desc(i):
        par = lax.rem(i, RING)
        return pltpu.make_async_copy(
            nodes_hbm.at[idx_v.at[pl.ds(i * GB, GB)]], buf.at[par],
            gsem.at[par])

      def w_desc(i):
        par = lax.rem(i, RING)
        return pltpu.make_async_copy(
            buf.at[par], out_hbm.at[pl.ds(base + i * GB, GB)], wsem.at[par])

      for j in range(RING):
        g_desc(j).start()

      def body(i, carry):
        g_desc(i).wait()
        w_desc(i).start()

        @pl.when(jnp.logical_and(i >= 1, i + RING - 1 < NCH))
        def _():
          w_desc(i - 1).wait()
          g_desc(i + RING - 1).start()

        return carry

      lax.fori_loop(0, NCH, body, 0)
      for j in range(RING):
        w_desc(NCH - RING + j).wait()

    run(idx_s, gs_hbm)
    run(idx_r, gr_hbm)

  return k(nodes_u, senders, receivers)


EPT = E // NS
SB = 80
NCHS = EPT // SB
NPAD = 10240
NPT = NPAD // NS
DH = D // NC
NLAST = N - (NS - 1) * NPT


def _sc_scatter(ue, receivers, zeros_half):
  @functools.partial(
      pl.kernel,
      out_type=jax.ShapeDtypeStruct((N, D), jnp.float32),
      mesh=_MESH(),
      scratch_types=[
          pltpu.VMEM((RING, SB), jnp.int32),
          pltpu.VMEM((RING, SB, DH), jnp.float32),
          pltpu.VMEM_SHARED((NPAD, DH), jnp.float32),
          pltpu.SemaphoreType.DMA((RING,)),
          pltpu.SemaphoreType.DMA((RING,)),
      ],
  )
  def k(ue_hbm, rcv_hbm, zero_hbm, agg_hbm, idx_v, rows_v, acc_sh, lsem,
        asem):
    c = lax.axis_index("c")
    s = lax.axis_index("s")
    col = pl.multiple_of(c * DH, DH)
    pltpu.sync_copy(zero_hbm, acc_sh.at[pl.ds(s * NPT, NPT)])
    plsc.subcore_barrier()
    ebase = s * EPT

    def idx_desc(i):
      par = lax.rem(i, RING)
      off = ebase + i * SB
      return pltpu.make_async_copy(rcv_hbm.at[pl.ds(off, SB)],
                                   idx_v.at[par], lsem.at[par])

    def rows_desc(i):
      par = lax.rem(i, RING)
      off = ebase + i * SB
      return pltpu.make_async_copy(
          ue_hbm.at[pl.ds(off, SB), pl.ds(col, DH)], rows_v.at[par],
          lsem.at[par])

    def add_start(i):
      par = lax.rem(i, RING)
      pltpu.async_copy(rows_v.at[par], acc_sh.at[idx_v.at[par]],
                       asem.at[par], add=True)

    def add_wait(i):
      par = lax.rem(i, RING)
      pltpu.make_async_copy(rows_v.at[par], acc_sh.at[idx_v.at[par]],
                            asem.at[par]).wait()

    def fire_load(i):
      idx_desc(i).start()
      rows_desc(i).start()

    for j in range(RING):
      fire_load(j)

    def body(i, carry):
      idx_desc(i).wait()
      rows_desc(i).wait()
      add_start(i)

      @pl.when(jnp.logical_and(i >= 1, i + RING - 1 < NCHS))
      def _():
        add_wait(i - 1)
        fire_load(i + RING - 1)

      return carry

    lax.fori_loop(0, NCHS, body, 0)
    for j in range(RING):
      add_wait(NCHS - RING + j)
    plsc.subcore_barrier()
    @pl.when(s < NS - 1)
    def _():
      pltpu.sync_copy(acc_sh.at[pl.ds(s * NPT, NPT)],
                      agg_hbm.at[pl.ds(s * NPT, NPT), pl.ds(col, DH)])

    @pl.when(s == NS - 1)
    def _():
      pltpu.sync_copy(acc_sh.at[pl.ds((NS - 1) * NPT, NLAST)],
                      agg_hbm.at[pl.ds((NS - 1) * NPT, NLAST),
                                 pl.ds(col, DH)])

  return k(ue, receivers, zeros_half)


BP = 1000


def _tc_pack(nodes):
  def body(n_r, out_r):
    xb = jax.lax.bitcast_convert_type(n_r[...], jnp.uint32)
    xb = xb + jnp.uint32(0x8000)
    out_r[...] = (xb[:, :DP] >> 16) | (xb[:, DP:] & jnp.uint32(0xFFFF0000))

  return pl.pallas_call(
      body,
      grid=(N // BP,),
      in_specs=[pl.BlockSpec((BP, D), lambda i: (i, 0))],
      out_specs=pl.BlockSpec((BP, DP), lambda i: (i, 0)),
      out_shape=jax.ShapeDtypeStruct((N, DP), jnp.uint32),
  )(nodes)


BE = 800


def _tc_edge_mlp(gs, gr, ed, ws, wr, we, b1, w2, b2):
  def body(gs_r, gr_r, ed_r, ws_r, wr_r, we_r, b1_r, w2_r, b2_r, out_r):
    bf = jnp.bfloat16
    hi = jnp.uint32(0xFFFF0000)

    def unpack(u):
      lo = jax.lax.bitcast_convert_type(u << 16, jnp.float32).astype(bf)
      up = jax.lax.bitcast_convert_type(u & hi, jnp.float32).astype(bf)
      return lo, up

    gsl, gsu = unpack(gs_r[...])
    grl, gru = unpack(gr_r[...])
    h = jnp.dot(gsl, ws_r[...][:DP], preferred_element_type=jnp.float32)
    h = h + jnp.dot(gsu, ws_r[...][DP:], preferred_element_type=jnp.float32)
    h = h + jnp.dot(grl, wr_r[...][:DP], preferred_element_type=jnp.float32)
    h = h + jnp.dot(gru, wr_r[...][DP:], preferred_element_type=jnp.float32)
    h = h + jnp.dot(ed_r[...], we_r[...], preferred_element_type=jnp.float32)
    h = jnp.maximum(h + b1_r[...], 0.0).astype(bf)
    out_r[...] = (jnp.dot(h, w2_r[...], preferred_element_type=jnp.float32)
                  + b2_r[...])

  full = lambda shape: pl.BlockSpec(shape, lambda i: (0, 0))
  return pl.pallas_call(
      body,
      grid=(E // BE,),
      in_specs=[
          pl.BlockSpec((BE, DP), lambda i: (i, 0)),
          pl.BlockSpec((BE, DP), lambda i: (i, 0)),
          pl.BlockSpec((BE, DE), lambda i: (i, 0)),
          full((D, H)),
          full((D, H)),
          full((DE, H)),
          full((1, H)),
          full((H, D)),
          full((1, D)),
      ],
      out_specs=pl.BlockSpec((BE, D), lambda i: (i, 0)),
      out_shape=jax.ShapeDtypeStruct((E, D), jnp.float32),
  )(gs, gr, ed, ws, wr, we, b1, w2, b2)


BN = 1000


def _tc_node_mlp(nodes, agg, w1a, w1b, b1, w2, b2):
  def body(n_r, a_r, w1a_r, w1b_r, b1_r, w2_r, b2_r, out_r):
    h = jnp.dot(n_r[...], w1a_r[...], preferred_element_type=jnp.float32)
    h = h + jnp.dot(a_r[...].astype(jnp.bfloat16), w1b_r[...],
                    preferred_element_type=jnp.float32)
    h = jnp.maximum(h + b1_r[...], 0.0).astype(jnp.bfloat16)
    out_r[...] = (jnp.dot(h, w2_r[...], preferred_element_type=jnp.float32)
                  + b2_r[...])

  full = lambda shape: pl.BlockSpec(shape, lambda i: (0, 0))
  return pl.pallas_call(
      body,
      grid=(N // BN,),
      in_specs=[
          pl.BlockSpec((BN, D), lambda i: (i, 0)),
          pl.BlockSpec((BN, D), lambda i: (i, 0)),
          full((D, H)),
          full((D, H)),
          full((1, H)),
          full((H, D)),
          full((1, D)),
      ],
      out_specs=pl.BlockSpec((BN, D), lambda i: (i, 0)),
      out_shape=jax.ShapeDtypeStruct((N, D), jnp.float32),
  )(nodes, agg, w1a, w1b, b1, w2, b2)


def kernel(nodes, edges, senders, receivers,
           We1, be1, We2, be2, Wn1, bn1, Wn2, bn2):
  bf = jnp.bfloat16
  We1b, We2b = We1.astype(bf), We2.astype(bf)
  ws, wr, we = We1b[:D], We1b[D:2 * D], We1b[2 * D:]
  nodes_u = _tc_pack(nodes)
  gs, gr = _sc_gather(nodes_u, senders, receivers)
  ue = _tc_edge_mlp(gs, gr, edges.astype(bf), ws, wr, we,
                    be1.reshape(1, H), We2b, be2.reshape(1, D))
  zeros_half = jnp.zeros((NPT, DH), jnp.float32)
  agg = _sc_scatter(ue, receivers, zeros_half)
  Wn1b, Wn2b = Wn1.astype(bf), Wn2.astype(bf)
  un = _tc_node_mlp(nodes.astype(bf), agg, Wn1b[:D], Wn1b[D:],
                    bn1.reshape(1, H), Wn2b, bn2.reshape(1, D))
  return (un, ue)


# speedup vs baseline: 3.2412x; 1.0689x over previous
import functools

import jax
import jax.numpy as jnp
from jax import lax
from jax.experimental import pallas as pl
from jax.experimental.pallas import tpu as pltpu
from jax.experimental.pallas import tpu_sc as plsc

N = 10000
E = 160000
D = 256
DE = 16
H = 512

NC = 2
NS = 16
NW = NC * NS

_MESH = lambda: plsc.VectorSubcoreMesh(
    core_axis_name="c", subcore_axis_name="s", num_cores=NC, num_subcores=NS)

EPW = E // NW
GB = 200


DP = D // 2


RING = 4
NCH = EPW // GB


def _sc_gather(nodes_u, senders, receivers):
  @functools.partial(
      pl.kernel,
      out_type=(jax.ShapeDtypeStruct((E, DP), jnp.uint32),
                jax.ShapeDtypeStruct((E, DP), jnp.uint32)),
      mesh=_MESH(),
      scratch_types=[
          pltpu.VMEM((EPW,), jnp.int32),
          pltpu.VMEM((EPW,), jnp.int32),
          pltpu.VMEM((RING, GB, DP), jnp.uint32),
          pltpu.SemaphoreType.DMA((RING,)),
          pltpu.SemaphoreType.DMA((RING,)),
      ],
  )
  def k(nodes_hbm, snd_hbm, rcv_hbm, gs_hbm, gr_hbm, idx_s, idx_r, buf,
        gsem, wsem):
    wid = lax.axis_index("s") * NC + lax.axis_index("c")
    base = wid * EPW
    pltpu.sync_copy(snd_hbm.at[pl.ds(base, EPW)], idx_s)
    pltpu.sync_copy(rcv_hbm.at[pl.ds(base, EPW)], idx_r)

    def run(idx_v, out_hbm):
      def g_desc(i):
        par = lax.rem(i, RING)
        return pltpu.make_async_copy(
            nodes_hbm.at[idx_v.at[pl.ds(i * GB, GB)]], buf.at[par],
            gsem.at[par])

      def w_desc(i):
        par = lax.rem(i, RING)
        return pltpu.make_async_copy(
            buf.at[par], out_hbm.at[pl.ds(base + i * GB, GB)], wsem.at[par])

      for j in range(RING):
        g_desc(j).start()

      def body(i, carry):
        g_desc(i).wait()
        w_desc(i).start()

        @pl.when(jnp.logical_and(i >= 1, i + RING - 1 < NCH))
        def _():
          w_desc(i - 1).wait()
          g_desc(i + RING - 1).start()

        return carry

      lax.fori_loop(0, NCH, body, 0)
      for j in range(RING):
        w_desc(NCH - RING + j).wait()

    run(idx_s, gs_hbm)
    run(idx_r, gr_hbm)

  return k(nodes_u, senders, receivers)


EPT = E // NS
SB = 80
NCHS = EPT // SB
NPAD = 10240
NPT = NPAD // NS
DH = D // NC
NLAST = N - (NS - 1) * NPT


def _sc_scatter(ue, receivers, zeros_half):
  @functools.partial(
      pl.kernel,
      out_type=jax.ShapeDtypeStruct((N, D), jnp.float32),
      mesh=_MESH(),
      scratch_types=[
          pltpu.VMEM((RING, SB), jnp.int32),
          pltpu.VMEM((RING, SB, DH), jnp.float32),
          pltpu.VMEM_SHARED((NPAD, DH), jnp.float32),
          pltpu.SemaphoreType.DMA((RING,)),
          pltpu.SemaphoreType.DMA((RING,)),
      ],
  )
  def k(ue_hbm, rcv_hbm, zero_hbm, agg_hbm, idx_v, rows_v, acc_sh, lsem,
        asem):
    c = lax.axis_index("c")
    s = lax.axis_index("s")
    col = pl.multiple_of(c * DH, DH)
    pltpu.sync_copy(zero_hbm, acc_sh.at[pl.ds(s * NPT, NPT)])
    plsc.subcore_barrier()
    ebase = s * EPT

    def idx_desc(i):
      par = lax.rem(i, RING)
      off = ebase + i * SB
      return pltpu.make_async_copy(rcv_hbm.at[pl.ds(off, SB)],
                                   idx_v.at[par], lsem.at[par])

    def rows_desc(i):
      par = lax.rem(i, RING)
      off = ebase + i * SB
      return pltpu.make_async_copy(
          ue_hbm.at[pl.ds(off, SB), pl.ds(col, DH)], rows_v.at[par],
          lsem.at[par])

    def add_start(i):
      par = lax.rem(i, RING)
      pltpu.async_copy(rows_v.at[par], acc_sh.at[idx_v.at[par]],
                       asem.at[par], add=True)

    def add_wait(i):
      par = lax.rem(i, RING)
      pltpu.make_async_copy(rows_v.at[par], acc_sh.at[idx_v.at[par]],
                            asem.at[par]).wait()

    def fire_load(i):
      idx_desc(i).start()
      rows_desc(i).start()

    for j in range(RING):
      fire_load(j)

    def body(i, carry):
      idx_desc(i).wait()
      rows_desc(i).wait()
      add_start(i)

      @pl.when(jnp.logical_and(i >= 1, i + RING - 1 < NCHS))
      def _():
        add_wait(i - 1)
        fire_load(i + RING - 1)

      return carry

    lax.fori_loop(0, NCHS, body, 0)
    for j in range(RING):
      add_wait(NCHS - RING + j)
    plsc.subcore_barrier()
    @pl.when(s < NS - 1)
    def _():
      pltpu.sync_copy(acc_sh.at[pl.ds(s * NPT, NPT)],
                      agg_hbm.at[pl.ds(s * NPT, NPT), pl.ds(col, DH)])

    @pl.when(s == NS - 1)
    def _():
      pltpu.sync_copy(acc_sh.at[pl.ds((NS - 1) * NPT, NLAST)],
                      agg_hbm.at[pl.ds((NS - 1) * NPT, NLAST),
                                 pl.ds(col, DH)])

  return k(ue, receivers, zeros_half)


BP = 1000


def _tc_pack(nodes):
  def body(n_r, out_r):
    xb = jax.lax.bitcast_convert_type(n_r[...], jnp.uint32)
    xb = xb + jnp.uint32(0x8000)
    out_r[...] = (xb[:, :DP] >> 16) | (xb[:, DP:] & jnp.uint32(0xFFFF0000))

  return pl.pallas_call(
      body,
      grid=(N // BP,),
      in_specs=[pl.BlockSpec((BP, D), lambda i: (i, 0))],
      out_specs=pl.BlockSpec((BP, DP), lambda i: (i, 0)),
      out_shape=jax.ShapeDtypeStruct((N, DP), jnp.uint32),
  )(nodes)


BE = 800


def _tc_edge_mlp(gs, gr, ed, w1, b1, w2, b2):
  def body(gs_r, gr_r, ed_r, w1_r, b1_r, w2_r, b2_r, out_r):
    bf = jnp.bfloat16
    hi = jnp.uint32(0xFFFF0000)

    def unpack(u):
      lo = jax.lax.bitcast_convert_type(u << 16, jnp.float32).astype(bf)
      up = jax.lax.bitcast_convert_type(u & hi, jnp.float32).astype(bf)
      return lo, up

    gsl, gsu = unpack(gs_r[...])
    grl, gru = unpack(gr_r[...])
    x = jnp.concatenate([gsl, gsu, grl, gru, ed_r[...]], axis=1)
    h = jnp.dot(x, w1_r[...], preferred_element_type=jnp.float32)
    h = jnp.maximum(h + b1_r[...], 0.0).astype(bf)
    out_r[...] = (jnp.dot(h, w2_r[...], preferred_element_type=jnp.float32)
                  + b2_r[...])

  full = lambda shape: pl.BlockSpec(shape, lambda i: (0, 0))
  return pl.pallas_call(
      body,
      grid=(E // BE,),
      in_specs=[
          pl.BlockSpec((BE, DP), lambda i: (i, 0)),
          pl.BlockSpec((BE, DP), lambda i: (i, 0)),
          pl.BlockSpec((BE, DE), lambda i: (i, 0)),
          full((2 * D + DE, H)),
          full((1, H)),
          full((H, D)),
          full((1, D)),
      ],
      out_specs=pl.BlockSpec((BE, D), lambda i: (i, 0)),
      out_shape=jax.ShapeDtypeStruct((E, D), jnp.float32),
  )(gs, gr, ed, w1, b1, w2, b2)


BN = 1000


def _tc_node_mlp(nodes, agg, w1a, w1b, b1, w2, b2):
  def body(n_r, a_r, w1a_r, w1b_r, b1_r, w2_r, b2_r, out_r):
    h = jnp.dot(n_r[...], w1a_r[...], preferred_element_type=jnp.float32)
    h = h + jnp.dot(a_r[...].astype(jnp.bfloat16), w1b_r[...],
                    preferred_element_type=jnp.float32)
    h = jnp.maximum(h + b1_r[...], 0.0).astype(jnp.bfloat16)
    out_r[...] = (jnp.dot(h, w2_r[...], preferred_element_type=jnp.float32)
                  + b2_r[...])

  full = lambda shape: pl.BlockSpec(shape, lambda i: (0, 0))
  return pl.pallas_call(
      body,
      grid=(N // BN,),
      in_specs=[
          pl.BlockSpec((BN, D), lambda i: (i, 0)),
          pl.BlockSpec((BN, D), lambda i: (i, 0)),
          full((D, H)),
          full((D, H)),
          full((1, H)),
          full((H, D)),
          full((1, D)),
      ],
      out_specs=pl.BlockSpec((BN, D), lambda i: (i, 0)),
      out_shape=jax.ShapeDtypeStruct((N, D), jnp.float32),
  )(nodes, agg, w1a, w1b, b1, w2, b2)


def kernel(nodes, edges, senders, receivers,
           We1, be1, We2, be2, Wn1, bn1, Wn2, bn2):
  bf = jnp.bfloat16
  We1b, We2b = We1.astype(bf), We2.astype(bf)
  nodes_u = _tc_pack(nodes)
  gs, gr = _sc_gather(nodes_u, senders, receivers)
  ue = _tc_edge_mlp(gs, gr, edges.astype(bf), We1b,
                    be1.reshape(1, H), We2b, be2.reshape(1, D))
  zeros_half = jnp.zeros((NPT, DH), jnp.float32)
  agg = _sc_scatter(ue, receivers, zeros_half)
  Wn1b, Wn2b = Wn1.astype(bf), Wn2.astype(bf)
  un = _tc_node_mlp(nodes.astype(bf), agg, Wn1b[:D], Wn1b[D:],
                    bn1.reshape(1, H), Wn2b, bn2.reshape(1, D))
  return (un, ue)
